# Initial kernel scaffold; baseline (speedup 1.0000x reference)
import jax
import jax.numpy as jnp
from jax.experimental import pallas as pl


def kernel(x, edge_index, W1, b1, g1, beta1, W2, b2, g2, beta2, W3, b3, g3, beta3):
    raise NotImplementedError("write your pallas kernel here")


# speedup vs baseline: 10.6937x; 10.6937x over previous
import functools

import jax
import jax.numpy as jnp
from jax import lax
from jax.experimental import pallas as pl
from jax.experimental.pallas import tpu as pltpu
from jax.experimental.pallas import tpu_sc as plsc

N = 10000
E = 320000
D = 128

NSC = 2
NTILE = 16
NW = NSC * NTILE
LANES = 16

C = 128
CHUNKS = -(-E // (NW * C))
PER_TILE = CHUNKS * C
E_PAD = PER_TILE * NW

N_PAD = 10240
ROWS_OUT = N_PAD // NTILE

_mesh = plsc.VectorSubcoreMesh(core_axis_name="c", subcore_axis_name="s")


@functools.partial(
    pl.kernel,
    out_type=jax.ShapeDtypeStruct((NSC, N_PAD, D), jnp.float32),
    mesh=_mesh,
    scratch_types=[
        pltpu.VMEM((CHUNKS, C), jnp.int32),
        pltpu.VMEM((C, D), jnp.float32),
        pltpu.VMEM((LANES, D), jnp.float32),
        pltpu.VMEM_SHARED((N_PAD, D), jnp.float32),
    ],
)
def _deg_kernel(dst_hbm, out_hbm, dstv, ones_v, zbuf, deg_sp):
    c = lax.axis_index("c")
    s = lax.axis_index("s")
    w = c * NTILE + s

    pltpu.sync_copy(dst_hbm.at[w], dstv)

    one = jnp.ones((LANES,), jnp.float32)
    zero = jnp.zeros((LANES,), jnp.float32)

    def fill(i, _):
        for j in range(D // LANES):
            ones_v[i, pl.ds(j * LANES, LANES)] = one
        return 0

    lax.fori_loop(0, C, fill, 0)

    def zfill(i, _):
        for j in range(D // LANES):
            zbuf[i, pl.ds(j * LANES, LANES)] = zero
        return 0

    lax.fori_loop(0, LANES, zfill, 0)

    base = s * ROWS_OUT

    def zrow(j, _):
        pltpu.sync_copy(zbuf, deg_sp.at[pl.ds(base + j * LANES, LANES)])
        return 0

    lax.fori_loop(0, ROWS_OUT // LANES, zrow, 0)

    plsc.subcore_barrier()

    def body(k, _):
        pltpu.sync_copy(ones_v, deg_sp.at[dstv.at[k]], add=True)
        return 0

    lax.fori_loop(0, CHUNKS, body, 0)

    plsc.subcore_barrier()

    pltpu.sync_copy(deg_sp.at[pl.ds(base, ROWS_OUT)],
                    out_hbm.at[c, pl.ds(base, ROWS_OUT)])


@functools.partial(
    pl.kernel,
    out_type=jax.ShapeDtypeStruct((NSC, N_PAD, D), jnp.float32),
    mesh=_mesh,
    scratch_types=[
        pltpu.VMEM((CHUNKS, C), jnp.int32),
        pltpu.VMEM((CHUNKS, C), jnp.int32),
        pltpu.VMEM((C, D), jnp.float32),
        pltpu.VMEM((LANES, D), jnp.float32),
        pltpu.SemaphoreType.DMA,
        pltpu.VMEM_SHARED((N_PAD, D), jnp.float32),
    ],
)
def _agg_kernel(hs_hbm, src_hbm, dst_hbm, out_hbm,
                srcv, dstv, buf, zbuf, sem, acc_sp):
    c = lax.axis_index("c")
    s = lax.axis_index("s")
    w = c * NTILE + s

    pltpu.sync_copy(src_hbm.at[w], srcv)
    pltpu.sync_copy(dst_hbm.at[w], dstv)

    zero = jnp.zeros((LANES,), jnp.float32)

    def zfill(i, _):
        for j in range(D // LANES):
            zbuf[i, pl.ds(j * LANES, LANES)] = zero
        return 0

    lax.fori_loop(0, LANES, zfill, 0)

    base = s * ROWS_OUT

    def zrow(j, _):
        pltpu.sync_copy(zbuf, acc_sp.at[pl.ds(base + j * LANES, LANES)])
        return 0

    lax.fori_loop(0, ROWS_OUT // LANES, zrow, 0)

    plsc.subcore_barrier()

    def body(k, _):
        pltpu.async_copy(hs_hbm.at[srcv.at[k]], buf, sem).wait()
        pltpu.sync_copy(buf, acc_sp.at[dstv.at[k]], add=True)
        return 0

    lax.fori_loop(0, CHUNKS, body, 0)

    plsc.subcore_barrier()

    pltpu.sync_copy(acc_sp.at[pl.ds(base, ROWS_OUT)],
                    out_hbm.at[c, pl.ds(base, ROWS_OUT)])


def _pre_body(x_ref, w_ref, degp_ref, hs_ref, dinv_ref):
    deg = degp_ref[0][:N, 0:1] + degp_ref[1][:N, 0:1] + 1.0
    dinv = lax.rsqrt(deg)
    h = jnp.dot(x_ref[...], w_ref[...], preferred_element_type=jnp.float32)
    hs_ref[...] = h * dinv
    dinv_ref[...] = dinv


def _combine(accp_ref, hs_ref, dinv_ref, b_ref, g_ref, be_ref):
    acc = accp_ref[0][:N] + accp_ref[1][:N]
    z = dinv_ref[...] * (acc + hs_ref[...]) + b_ref[...]
    mu = jnp.mean(z, axis=-1, keepdims=True)
    zc = z - mu
    var = jnp.mean(zc * zc, axis=-1, keepdims=True)
    return zc * lax.rsqrt(var + 1e-5) * g_ref[...] + be_ref[...]


def _mid_body(accp_ref, hs_ref, dinv_ref, b_ref, g_ref, be_ref, w_ref,
              out_ref):
    z = jnp.maximum(_combine(accp_ref, hs_ref, dinv_ref, b_ref, g_ref,
                             be_ref), 0.0)
    h = jnp.dot(z, w_ref[...], preferred_element_type=jnp.float32)
    out_ref[...] = h * dinv_ref[...]


def _fin_body(accp_ref, hs_ref, dinv_ref, b_ref, g_ref, be_ref, out_ref):
    out_ref[...] = _combine(accp_ref, hs_ref, dinv_ref, b_ref, g_ref, be_ref)


_pre_call = pl.pallas_call(
    _pre_body,
    out_shape=[
        jax.ShapeDtypeStruct((N, D), jnp.float32),
        jax.ShapeDtypeStruct((N, 1), jnp.float32),
    ],
)

_mid_call = pl.pallas_call(
    _mid_body,
    out_shape=jax.ShapeDtypeStruct((N, D), jnp.float32),
)

_fin_call = pl.pallas_call(
    _fin_body,
    out_shape=jax.ShapeDtypeStruct((N, D), jnp.float32),
)


def kernel(x, edge_index, W1, b1, g1, beta1, W2, b2, g2, beta2,
           W3, b3, g3, beta3):
    src = edge_index[0]
    dst = edge_index[1]

    pad = E_PAD - E
    src_w = jnp.concatenate([src, jnp.zeros((pad,), jnp.int32)])
    src_w = src_w.reshape(NW, CHUNKS, C)
    dst_w = jnp.concatenate([dst, jnp.full((pad,), N, jnp.int32)])
    dst_w = dst_w.reshape(NW, CHUNKS, C)

    b1r, g1r, be1 = b1.reshape(1, D), g1.reshape(1, D), beta1.reshape(1, D)
    b2r, g2r, be2 = b2.reshape(1, D), g2.reshape(1, D), beta2.reshape(1, D)
    b3r, g3r, be3 = b3.reshape(1, D), g3.reshape(1, D), beta3.reshape(1, D)

    degp = _deg_kernel(dst_w)
    hs, dinv = _pre_call(x, W1, degp)
    accp = _agg_kernel(hs, src_w, dst_w)
    hs = _mid_call(accp, hs, dinv, b1r, g1r, be1, W2)
    accp = _agg_kernel(hs, src_w, dst_w)
    hs = _mid_call(accp, hs, dinv, b2r, g2r, be2, W3)
    accp = _agg_kernel(hs, src_w, dst_w)
    return _fin_call(accp, hs, dinv, b3r, g3r, be3)
